# baseline (speedup 1.0000x reference)
import functools

import jax
import jax.numpy as jnp
from jax import lax
from jax.experimental import pallas as pl
from jax.experimental.pallas import tpu as pltpu
from jax.experimental.pallas import tpu_sc as plsc

BATCH = 4096
EMBED_DIM = 32
NUM_CORES = 2
NUM_SUBCORES = 16
NUM_WORKERS = NUM_CORES * NUM_SUBCORES
B_PER_W = BATCH // NUM_WORKERS
L = 16
GROUPS = B_PER_W // L
NBUF = 4
BLK = 128


def _make_user_kernel():
    mesh = plsc.VectorSubcoreMesh(core_axis_name="c", subcore_axis_name="s")

    @functools.partial(
        pl.kernel,
        mesh=mesh,
        out_type=jax.ShapeDtypeStruct((EMBED_DIM, BATCH), jnp.float32),
        compiler_params=pltpu.CompilerParams(needs_layout_passes=False),
        scratch_types=[
            pltpu.VMEM((B_PER_W,), jnp.int32),
            pltpu.VMEM((NBUF, EMBED_DIM, BLK), jnp.float32),
            pltpu.VMEM((EMBED_DIM, B_PER_W), jnp.float32),
        ]
        + [pltpu.SemaphoreType.DMA] * NBUF,
    )
    def lookup(uid_hbm, w_hbm, out_hbm, idx_v, bounce, col_v, *sems):
        wid = lax.axis_index("s") * NUM_CORES + lax.axis_index("c")
        base = wid * B_PER_W
        pltpu.sync_copy(uid_hbm.at[pl.ds(base, B_PER_W)], idx_v)

        cvec0 = lax.iota(jnp.int32, L)
        cvec1 = cvec0 + L

        pending = []

        def extract(slot, j, roff):
            roffv = jnp.full((L,), roff, jnp.int32)
            jv = jnp.full((L,), j, jnp.int32)
            top = plsc.load_gather(bounce.at[slot], [cvec0, roffv])
            bot = plsc.load_gather(bounce.at[slot], [cvec1, roffv])
            plsc.store_scatter(col_v, [cvec0, jv], top)
            plsc.store_scatter(col_v, [cvec1, jv], bot)

        for g in range(GROUPS):
            rv = idx_v[pl.ds(g * L, L)]
            for l in range(L):
                j = g * L + l
                r = rv[l]
                rb = pl.multiple_of(lax.shift_left(
                    lax.shift_right_logical(r, 7), 7), BLK)
                roff = lax.bitwise_and(r, BLK - 1)
                slot = j % NBUF
                if len(pending) == NBUF:
                    cp, pslot, pj, proff = pending.pop(0)
                    cp.wait()
                    extract(pslot, pj, proff)
                cp = pltpu.async_copy(
                    w_hbm.at[:, pl.ds(rb, BLK)], bounce.at[slot], sems[slot]
                )
                pending.append((cp, slot, j, roff))
        for cp, pslot, pj, proff in pending:
            cp.wait()
            extract(pslot, pj, proff)

        pltpu.sync_copy(col_v, out_hbm.at[:, pl.ds(base, B_PER_W)])

    return lookup


def _make_small_tables_kernel():
    mesh = plsc.VectorSubcoreMesh(core_axis_name="c", subcore_axis_name="s")

    @functools.partial(
        pl.kernel,
        mesh=mesh,
        out_type=(
            jax.ShapeDtypeStruct((BATCH, EMBED_DIM), jnp.float32),
            jax.ShapeDtypeStruct((BATCH, EMBED_DIM), jnp.float32),
        ),
        compiler_params=pltpu.CompilerParams(use_tc_tiling_on_sc=False),
        scratch_types=[
            pltpu.VMEM((B_PER_W,), jnp.int32),
            pltpu.VMEM((B_PER_W,), jnp.int32),
            pltpu.VMEM((B_PER_W, EMBED_DIM), jnp.float32),
            pltpu.VMEM((B_PER_W, EMBED_DIM), jnp.float32),
            pltpu.SemaphoreType.DMA,
        ],
    )
    def lookup(iid_hbm, cid_hbm, wi_hbm, wc_hbm, out_i, out_c,
               idx_i, idx_c, rows_i, rows_c, sem):
        wid = lax.axis_index("s") * NUM_CORES + lax.axis_index("c")
        base = wid * B_PER_W
        pltpu.sync_copy(iid_hbm.at[pl.ds(base, B_PER_W)], idx_i)
        pltpu.sync_copy(cid_hbm.at[pl.ds(base, B_PER_W)], idx_c)
        ci = pltpu.async_copy(wi_hbm.at[idx_i], rows_i, sem)
        cc = pltpu.async_copy(wc_hbm.at[idx_c], rows_c, sem)
        ci.wait()
        cc.wait()
        pltpu.sync_copy(rows_i, out_i.at[pl.ds(base, B_PER_W)])
        pltpu.sync_copy(rows_c, out_c.at[pl.ds(base, B_PER_W)])

    return lookup


_user_lookup = _make_user_kernel()
_small_lookup = _make_small_tables_kernel()


def kernel(user_id, item_id, category, W_user, W_item, W_category):
    out_u = _user_lookup(user_id.astype(jnp.int32), W_user.T)
    out_i, out_c = _small_lookup(
        item_id.astype(jnp.int32),
        category.astype(jnp.int32),
        W_item,
        W_category,
    )
    return (out_u.T, out_i, out_c)
